# baseline (speedup 1.0000x reference)
import jax
import jax.numpy as jnp
from jax.experimental import pallas as pl

_B, _D, _H, _A = 16384, 500, 500, 8
_BT = 512
_LANES = 128


def _fused_body(x_ref, u_ref, w1_ref, b1_ref, w2_ref, b2_ref, out_ref):
    x = x_ref[...]
    h = jnp.dot(x, w1_ref[...], preferred_element_type=jnp.float32)
    h = jnp.maximum(h + b1_ref[...], 0.0)
    z = jnp.dot(h, w2_ref[...], preferred_element_type=jnp.float32)
    z = z + b2_ref[...]
    lane = jax.lax.broadcasted_iota(jnp.int32, z.shape, 1)
    zm = jnp.where(lane < _A, z, -1e30)
    m = jnp.max(zm, axis=1, keepdims=True)
    e = jnp.exp(zm - m)
    prob = e / jnp.sum(e, axis=1, keepdims=True)
    logits = jnp.log(prob + 1e-20)
    logits = jnp.where(lane < _A, logits, -1e30)
    m2 = jnp.max(logits, axis=1, keepdims=True)
    pdf = jnp.where(lane < _A, jnp.exp(logits - m2), 0.0)
    tri = (jax.lax.broadcasted_iota(jnp.int32, (_LANES, _LANES), 0)
           <= jax.lax.broadcasted_iota(jnp.int32, (_LANES, _LANES), 1)
           ).astype(jnp.float32)
    cdf = jnp.dot(pdf, tri, preferred_element_type=jnp.float32,
                  precision=jax.lax.Precision.HIGHEST)
    total = jnp.max(cdf, axis=1, keepdims=True)
    us = u_ref[...] * total
    cnt = jnp.sum(jnp.where((lane < _A) & (cdf <= us), 1, 0), axis=1)
    out_ref[...] = cnt.astype(jnp.int32)[:, None]


def kernel(inputs, u, W1, b1, W2, b2):
    w2p = jnp.zeros((_D, _LANES), dtype=jnp.float32).at[:, :_A].set(W2)
    b2p = jnp.zeros((1, _LANES), dtype=jnp.float32).at[0, :_A].set(b2)
    b1r = b1.reshape(1, _H)
    grid = (_B // _BT,)
    out = pl.pallas_call(
        _fused_body,
        grid=grid,
        in_specs=[
            pl.BlockSpec((_BT, _D), lambda i: (i, 0)),
            pl.BlockSpec((_BT, 1), lambda i: (i, 0)),
            pl.BlockSpec((_D, _H), lambda i: (0, 0)),
            pl.BlockSpec((1, _H), lambda i: (0, 0)),
            pl.BlockSpec((_D, _LANES), lambda i: (0, 0)),
            pl.BlockSpec((1, _LANES), lambda i: (0, 0)),
        ],
        out_specs=pl.BlockSpec((_BT, 1), lambda i: (i, 0)),
        out_shape=jax.ShapeDtypeStruct((_B, 1), jnp.int32),
    )(inputs, u, W1, b1r, w2p, b2p)
    return out.reshape(_B).astype(jnp.int64)
